# baseline (speedup 1.0000x reference)
import functools

import jax
import jax.numpy as jnp
from jax import lax
from jax.experimental import pallas as pl
from jax.experimental.pallas import tpu as pltpu
from jax.experimental.pallas import tpu_sc as plsc

N = 10000
E = 320000
D = 128
ALPHA_T = 0.1
EPS = 1e-8

NC = 2
NS = 16
L = 16
NW = NC * NS
C = 40
G8 = 8


def _tc_normalize(x):
    rb = 2000

    def body(x_ref, y_ref):
        xb = x_ref[...]
        nrm = jnp.maximum(jnp.sqrt(jnp.sum(xb * xb, axis=1, keepdims=True)), EPS)
        y_ref[...] = xb / nrm

    return pl.pallas_call(
        body,
        out_shape=jax.ShapeDtypeStruct((N, D), jnp.float32),
        grid=(N // rb,),
        in_specs=[pl.BlockSpec((rb, D), lambda i: (i, 0))],
        out_specs=pl.BlockSpec((rb, D), lambda i: (i, 0)),
    )(x)


def _tc_sum2(q0, q1):
    rb = 2000

    def body(a_ref, b_ref, o_ref):
        o_ref[...] = a_ref[...] + b_ref[...]

    return pl.pallas_call(
        body,
        out_shape=jax.ShapeDtypeStruct((N, D), jnp.float32),
        grid=(N // rb,),
        in_specs=[pl.BlockSpec((rb, D), lambda i: (i, 0))] * 2,
        out_specs=pl.BlockSpec((rb, D), lambda i: (i, 0)),
    )(q0, q1)


def _tc_combine_norm(p0, p1, xprev):
    rb = 2000

    def body(a_ref, b_ref, c_ref, x_ref, y_ref):
        xn = a_ref[...] + b_ref[...] + c_ref[...]
        x_ref[...] = xn
        nrm = jnp.maximum(jnp.sqrt(jnp.sum(xn * xn, axis=1, keepdims=True)), EPS)
        y_ref[...] = xn / nrm

    return pl.pallas_call(
        body,
        out_shape=[jax.ShapeDtypeStruct((N, D), jnp.float32)] * 2,
        grid=(N // rb,),
        in_specs=[pl.BlockSpec((rb, D), lambda i: (i, 0))] * 3,
        out_specs=[pl.BlockSpec((rb, D), lambda i: (i, 0))] * 2,
    )(p0, p1, xprev)


def _tc_final(p0, p1, x0, x1):
    rb = 2000

    def body(a_ref, b_ref, c_ref, d_ref, o_ref):
        o_ref[...] = (a_ref[...] + b_ref[...] + c_ref[...]
                      + 2.0 * d_ref[...]) * (1.0 / 3.0)

    return pl.pallas_call(
        body,
        out_shape=jax.ShapeDtypeStruct((N, D), jnp.float32),
        grid=(N // rb,),
        in_specs=[pl.BlockSpec((rb, D), lambda i: (i, 0))] * 4,
        out_specs=pl.BlockSpec((rb, D), lambda i: (i, 0)),
    )(p0, p1, x0, x1)


def _sc_pass(ytab, mtab, e0, e1, vals, zeros):
    per_w = E // NW
    chunks = per_w // C
    r0 = (-(-N // NS) + 7) // 8 * 8
    r1 = N - (NS - 1) * r0

    mesh = plsc.VectorSubcoreMesh(core_axis_name="c", subcore_axis_name="s",
                                  num_cores=NC, num_subcores=NS)
    SL = 3

    @functools.partial(
        pl.kernel,
        out_type=jax.ShapeDtypeStruct((NC, N, D), jnp.float32),
        mesh=mesh,
        scratch_types=(
            [pltpu.VMEM((C,), jnp.int32)] * SL
            + [pltpu.VMEM((C,), jnp.int32)] * SL
            + [pltpu.VMEM((C + L,), jnp.float32)] * SL
            + [pltpu.VMEM((C, D), jnp.float32)] * SL
            + [pltpu.VMEM((C, D), jnp.float32)] * SL
            + [pltpu.VMEM((C, D), jnp.float32)] * SL
            + [pltpu.VMEM((C,), jnp.int32)] * SL
            + [pltpu.VMEM((G8, 24), jnp.float32)]
            + [pltpu.VMEM_SHARED((N, D), jnp.float32)]
            + [pltpu.SemaphoreType.DMA] * (3 * SL)
        ),
    )
    def body(ytab_h, mtab_h, e0_h, e1_h, vals_h, zeros_h, out_h, *refs):
        i0 = refs[0:SL]
        i1 = refs[SL:2 * SL]
        va = refs[2 * SL:3 * SL]
        rr0 = refs[3 * SL:4 * SL]
        rr1 = refs[4 * SL:5 * SL]
        rm = refs[5 * SL:6 * SL]
        io_sc = refs[6 * SL:7 * SL]
        fold = refs[7 * SL]
        acc_sh = refs[7 * SL + 1]
        sem_l = refs[7 * SL + 2:7 * SL + 2 + SL]
        sem_g = refs[7 * SL + 2 + SL:7 * SL + 2 + 2 * SL]
        sem_s = refs[7 * SL + 2 + 2 * SL:7 * SL + 2 + 3 * SL]

        cid = lax.axis_index("c")
        sid = lax.axis_index("s")
        wid = cid * NS + sid

        @pl.when(sid < NS - 1)
        def _():
            pltpu.sync_copy(zeros_h.at[pl.ds(sid * r0, r0)],
                            acc_sh.at[pl.ds(sid * r0, r0)])

        @pl.when(sid == NS - 1)
        def _():
            pltpu.sync_copy(zeros_h.at[pl.ds((NS - 1) * r0, r1)],
                            acc_sh.at[pl.ds((NS - 1) * r0, r1)])

        plsc.subcore_barrier()

        base = wid * per_w

        def load_descs(x, s):
            off = base + x * C
            return [(e0_h.at[pl.ds(off, C)], i0[s], sem_l[s]),
                    (e1_h.at[pl.ds(off, C)], i1[s], sem_l[s]),
                    (vals_h.at[pl.ds(off, C)], va[s].at[pl.ds(0, C)], sem_l[s])]

        def gather_descs(s):
            return [(ytab_h.at[i0[s]], rr0[s], sem_g[s]),
                    (ytab_h.at[i1[s]], rr1[s], sem_g[s]),
                    (mtab_h.at[i1[s]], rm[s], sem_g[s])]

        def wait_scatter(s):
            pltpu.make_async_copy(rm[s], acc_sh.at[io_sc[s]], sem_s[s]).wait()

        def issue_loads(x, s):
            for src, dst, sm in load_descs(x, s):
                pltpu.async_copy(src, dst, sm)

        def issue_gathers(x, s, wait_sc=True):
            for src, dst, sm in load_descs(x, s):
                pltpu.make_async_copy(src, dst, sm).wait()
            gd = gather_descs(s)
            for src, dst, sm in gd[:2]:
                pltpu.async_copy(src, dst, sm)
            if wait_sc:
                wait_scatter(s)
            src, dst, sm = gd[2]
            pltpu.async_copy(src, dst, sm)

        def run_chunk(s):
            for src, dst, sm in gather_descs(s):
                pltpu.make_async_copy(src, dst, sm).wait()
            r0v, r1v, rmv = rr0[s], rr1[s], rm[s]

            def group_body(g, carry2):
                val16 = va[s][pl.ds(g * G8, L)]
                for i in range(G8):
                    c = g * G8 + i
                    acc = r0v[c, pl.ds(0, L)] * r1v[c, pl.ds(0, L)]
                    for jj in range(1, D // L):
                        acc = acc + (r0v[c, pl.ds(jj * L, L)]
                                     * r1v[c, pl.ds(jj * L, L)])
                    fold[i, pl.ds(0, L)] = acc
                    acc = acc + fold[i, pl.ds(G8, L)]
                    e = [acc[k] for k in range(G8)]
                    while len(e) > 1:
                        e = [e[t] + e[t + 1] for t in range(0, len(e), 2)]
                    simn = jnp.clip((e[0] + 1.0) * 0.5, 0.0, 1.0)
                    w = val16[i] * (1.0 + ALPHA_T * simn)
                    for jj in range(D // L):
                        rmv[c, pl.ds(jj * L, L)] = rmv[c, pl.ds(jj * L, L)] * w
                return carry2

            lax.fori_loop(0, C // G8, group_body, 0)
            io_sc[s][pl.ds(0, L)] = i0[s][pl.ds(0, L)]
            io_sc[s][pl.ds(L, L)] = i0[s][pl.ds(L, L)]
            io_sc[s][pl.ds(C - L, L)] = i0[s][pl.ds(C - L, L)]
            pltpu.async_copy(rm[s], acc_sh.at[io_sc[s]], sem_s[s], add=True)

        issue_loads(0, 0)
        issue_gathers(0, 0, wait_sc=False)
        issue_loads(1, 1)

        issue_gathers(1, 1, wait_sc=False)
        issue_loads(2, 2)
        run_chunk(0)
        issue_gathers(2, 2, wait_sc=False)
        issue_loads(3, 0)
        run_chunk(1)
        issue_gathers(3, 0)
        issue_loads(4, 1)
        run_chunk(2)

        def loop_body(k, carry):
            a = SL * k
            issue_gathers(a + 1, 1)
            issue_loads(a + 2, 2)
            run_chunk(0)
            issue_gathers(a + 2, 2)
            issue_loads(a + 3, 0)
            run_chunk(1)
            issue_gathers(a + 3, 0)
            issue_loads(a + 4, 1)
            run_chunk(2)
            return carry

        lax.fori_loop(1, chunks // SL, loop_body, 0)

        run_chunk(0)
        for src, dst, sm in load_descs(chunks, 1):
            pltpu.make_async_copy(src, dst, sm).wait()
        wait_scatter(1)
        wait_scatter(2)
        wait_scatter(0)

        plsc.subcore_barrier()

        @pl.when(sid < NS - 1)
        def _():
            pltpu.sync_copy(acc_sh.at[pl.ds(sid * r0, r0)],
                            out_h.at[cid, pl.ds(sid * r0, r0)])

        @pl.when(sid == NS - 1)
        def _():
            pltpu.sync_copy(acc_sh.at[pl.ds((NS - 1) * r0, r1)],
                            out_h.at[cid, pl.ds((NS - 1) * r0, r1)])

    return body(ytab, mtab, e0, e1, vals, zeros)


def kernel(pois_embs, src_edge_index, src_values, tar_edge_index, tar_values):
    x0 = pois_embs
    zeros = jnp.zeros((N, D), jnp.float32)
    zi = jnp.zeros((64,), jnp.int32)
    zf = jnp.zeros((64,), jnp.float32)
    t0 = jnp.concatenate([tar_edge_index[0], zi])
    t1 = jnp.concatenate([tar_edge_index[1], zi])
    s0 = jnp.concatenate([src_edge_index[0], zi])
    s1 = jnp.concatenate([src_edge_index[1], zi])
    tv = jnp.concatenate([tar_values, zf])
    sv = jnp.concatenate([src_values, zf])

    y0 = _tc_normalize(x0)
    q = _sc_pass(y0, x0, t0, t1, tv, zeros)
    mt = _tc_sum2(q[0], q[1])
    p = _sc_pass(y0, mt, s0, s1, sv, zeros)
    x1, y1 = _tc_combine_norm(p[0], p[1], x0)

    q = _sc_pass(y1, x1, t0, t1, tv, zeros)
    mt = _tc_sum2(q[0], q[1])
    p = _sc_pass(y1, mt, s0, s1, sv, zeros)
    return _tc_final(p[0], p[1], x0, x1)


# speedup vs baseline: 1.3268x; 1.3268x over previous
import functools

import jax
import jax.numpy as jnp
from jax import lax
from jax.experimental import pallas as pl
from jax.experimental.pallas import tpu as pltpu
from jax.experimental.pallas import tpu_sc as plsc

N = 10000
E = 320000
D = 128
ALPHA_T = 0.1
EPS = 1e-8

NC = 2
NS = 16
L = 16
NW = NC * NS
C = 40
G8 = 8


def _tc_normalize(x):
    rb = 2000

    def body(x_ref, y_ref):
        xb = x_ref[...]
        nrm = jnp.maximum(jnp.sqrt(jnp.sum(xb * xb, axis=1, keepdims=True)), EPS)
        y_ref[...] = xb / nrm

    return pl.pallas_call(
        body,
        out_shape=jax.ShapeDtypeStruct((N, D), jnp.float32),
        grid=(N // rb,),
        in_specs=[pl.BlockSpec((rb, D), lambda i: (i, 0))],
        out_specs=pl.BlockSpec((rb, D), lambda i: (i, 0)),
    )(x)


def _tc_sum2(q0, q1):
    rb = 2000

    def body(a_ref, b_ref, o_ref):
        o_ref[...] = a_ref[...] + b_ref[...]

    return pl.pallas_call(
        body,
        out_shape=jax.ShapeDtypeStruct((N, D), jnp.float32),
        grid=(N // rb,),
        in_specs=[pl.BlockSpec((rb, D), lambda i: (i, 0))] * 2,
        out_specs=pl.BlockSpec((rb, D), lambda i: (i, 0)),
    )(q0, q1)


def _tc_combine_norm(p0, p1, xprev):
    rb = 2000

    def body(a_ref, b_ref, c_ref, x_ref, y_ref):
        xn = a_ref[...] + b_ref[...] + c_ref[...]
        x_ref[...] = xn
        nrm = jnp.maximum(jnp.sqrt(jnp.sum(xn * xn, axis=1, keepdims=True)), EPS)
        y_ref[...] = xn / nrm

    return pl.pallas_call(
        body,
        out_shape=[jax.ShapeDtypeStruct((N, D), jnp.float32)] * 2,
        grid=(N // rb,),
        in_specs=[pl.BlockSpec((rb, D), lambda i: (i, 0))] * 3,
        out_specs=[pl.BlockSpec((rb, D), lambda i: (i, 0))] * 2,
    )(p0, p1, xprev)


def _tc_final(p0, p1, x0, x1):
    rb = 2000

    def body(a_ref, b_ref, c_ref, d_ref, o_ref):
        o_ref[...] = (a_ref[...] + b_ref[...] + c_ref[...]
                      + 2.0 * d_ref[...]) * (1.0 / 3.0)

    return pl.pallas_call(
        body,
        out_shape=jax.ShapeDtypeStruct((N, D), jnp.float32),
        grid=(N // rb,),
        in_specs=[pl.BlockSpec((rb, D), lambda i: (i, 0))] * 4,
        out_specs=pl.BlockSpec((rb, D), lambda i: (i, 0)),
    )(p0, p1, x0, x1)


def _sc_pass(ytab, mtab, e0, e1, vals, zeros):
    per_w = E // NW
    chunks = per_w // C
    r0 = (-(-N // NS) + 7) // 8 * 8
    r1 = N - (NS - 1) * r0

    mesh = plsc.VectorSubcoreMesh(core_axis_name="c", subcore_axis_name="s",
                                  num_cores=NC, num_subcores=NS)
    SL = 3

    @functools.partial(
        pl.kernel,
        out_type=jax.ShapeDtypeStruct((NC, N, D), jnp.float32),
        mesh=mesh,
        scratch_types=(
            [pltpu.VMEM((C,), jnp.int32)] * SL
            + [pltpu.VMEM((C,), jnp.int32)] * SL
            + [pltpu.VMEM((C + L,), jnp.float32)] * SL
            + [pltpu.VMEM((C, D), jnp.float32)] * SL
            + [pltpu.VMEM((C, D), jnp.float32)] * SL
            + [pltpu.VMEM((C, D), jnp.float32)] * SL
            + [pltpu.VMEM((C,), jnp.int32)] * SL
            + [pltpu.VMEM_SHARED((N, D), jnp.float32)]
            + [pltpu.SemaphoreType.DMA] * (3 * SL)
        ),
    )
    def body(ytab_h, mtab_h, e0_h, e1_h, vals_h, zeros_h, out_h, *refs):
        i0 = refs[0:SL]
        i1 = refs[SL:2 * SL]
        va = refs[2 * SL:3 * SL]
        rr0 = refs[3 * SL:4 * SL]
        rr1 = refs[4 * SL:5 * SL]
        rm = refs[5 * SL:6 * SL]
        io_sc = refs[6 * SL:7 * SL]
        acc_sh = refs[7 * SL]
        sem_l = refs[7 * SL + 1:7 * SL + 1 + SL]
        sem_g = refs[7 * SL + 1 + SL:7 * SL + 1 + 2 * SL]
        sem_s = refs[7 * SL + 1 + 2 * SL:7 * SL + 1 + 3 * SL]

        cid = lax.axis_index("c")
        sid = lax.axis_index("s")
        wid = cid * NS + sid

        @pl.when(sid < NS - 1)
        def _():
            pltpu.sync_copy(zeros_h.at[pl.ds(sid * r0, r0)],
                            acc_sh.at[pl.ds(sid * r0, r0)])

        @pl.when(sid == NS - 1)
        def _():
            pltpu.sync_copy(zeros_h.at[pl.ds((NS - 1) * r0, r1)],
                            acc_sh.at[pl.ds((NS - 1) * r0, r1)])

        plsc.subcore_barrier()

        base = wid * per_w

        def load_descs(x, s):
            off = base + x * C
            return [(e0_h.at[pl.ds(off, C)], i0[s], sem_l[s]),
                    (e1_h.at[pl.ds(off, C)], i1[s], sem_l[s]),
                    (vals_h.at[pl.ds(off, C)], va[s].at[pl.ds(0, C)], sem_l[s])]

        def gather_descs(s):
            return [(ytab_h.at[i0[s]], rr0[s], sem_g[s]),
                    (ytab_h.at[i1[s]], rr1[s], sem_g[s]),
                    (mtab_h.at[i1[s]], rm[s], sem_g[s])]

        def wait_scatter(s):
            pltpu.make_async_copy(rm[s], acc_sh.at[io_sc[s]], sem_s[s]).wait()

        def issue_loads(x, s):
            for src, dst, sm in load_descs(x, s):
                pltpu.async_copy(src, dst, sm)

        def issue_gathers(x, s, wait_sc=True):
            for src, dst, sm in load_descs(x, s):
                pltpu.make_async_copy(src, dst, sm).wait()
            gd = gather_descs(s)
            for src, dst, sm in gd[:2]:
                pltpu.async_copy(src, dst, sm)
            if wait_sc:
                wait_scatter(s)
            src, dst, sm = gd[2]
            pltpu.async_copy(src, dst, sm)

        def run_chunk(s):
            for src, dst, sm in gather_descs(s):
                pltpu.make_async_copy(src, dst, sm).wait()
            r0v, r1v, rmv = rr0[s], rr1[s], rm[s]

            def group_body(g, carry2):
                val16 = va[s][pl.ds(g * G8, L)]
                for i in range(G8):
                    c = g * G8 + i
                    acc = r0v[c, pl.ds(0, L)] * r1v[c, pl.ds(0, L)]
                    for jj in range(1, D // L):
                        acc = acc + (r0v[c, pl.ds(jj * L, L)]
                                     * r1v[c, pl.ds(jj * L, L)])
                    e = [acc[k] for k in range(L)]
                    while len(e) > 1:
                        e = [e[t] + e[t + 1] for t in range(0, len(e), 2)]
                    simn = jnp.clip((e[0] + 1.0) * 0.5, 0.0, 1.0)
                    w = val16[i] * (1.0 + ALPHA_T * simn)
                    for jj in range(D // L):
                        rmv[c, pl.ds(jj * L, L)] = rmv[c, pl.ds(jj * L, L)] * w
                return carry2

            lax.fori_loop(0, C // G8, group_body, 0)
            io_sc[s][pl.ds(0, L)] = i0[s][pl.ds(0, L)]
            io_sc[s][pl.ds(L, L)] = i0[s][pl.ds(L, L)]
            io_sc[s][pl.ds(C - L, L)] = i0[s][pl.ds(C - L, L)]
            pltpu.async_copy(rm[s], acc_sh.at[io_sc[s]], sem_s[s], add=True)

        issue_loads(0, 0)
        issue_gathers(0, 0, wait_sc=False)
        issue_loads(1, 1)

        issue_gathers(1, 1, wait_sc=False)
        issue_loads(2, 2)
        run_chunk(0)
        issue_gathers(2, 2, wait_sc=False)
        issue_loads(3, 0)
        run_chunk(1)
        issue_gathers(3, 0)
        issue_loads(4, 1)
        run_chunk(2)

        def loop_body(k, carry):
            a = SL * k
            issue_gathers(a + 1, 1)
            issue_loads(a + 2, 2)
            run_chunk(0)
            issue_gathers(a + 2, 2)
            issue_loads(a + 3, 0)
            run_chunk(1)
            issue_gathers(a + 3, 0)
            issue_loads(a + 4, 1)
            run_chunk(2)
            return carry

        lax.fori_loop(1, chunks // SL, loop_body, 0)

        run_chunk(0)
        for src, dst, sm in load_descs(chunks, 1):
            pltpu.make_async_copy(src, dst, sm).wait()
        wait_scatter(1)
        wait_scatter(2)
        wait_scatter(0)

        plsc.subcore_barrier()

        @pl.when(sid < NS - 1)
        def _():
            pltpu.sync_copy(acc_sh.at[pl.ds(sid * r0, r0)],
                            out_h.at[cid, pl.ds(sid * r0, r0)])

        @pl.when(sid == NS - 1)
        def _():
            pltpu.sync_copy(acc_sh.at[pl.ds((NS - 1) * r0, r1)],
                            out_h.at[cid, pl.ds((NS - 1) * r0, r1)])

    return body(ytab, mtab, e0, e1, vals, zeros)


def kernel(pois_embs, src_edge_index, src_values, tar_edge_index, tar_values):
    x0 = pois_embs
    zeros = jnp.zeros((N, D), jnp.float32)
    zi = jnp.zeros((64,), jnp.int32)
    zf = jnp.zeros((64,), jnp.float32)
    t0 = jnp.concatenate([tar_edge_index[0], zi])
    t1 = jnp.concatenate([tar_edge_index[1], zi])
    s0 = jnp.concatenate([src_edge_index[0], zi])
    s1 = jnp.concatenate([src_edge_index[1], zi])
    tv = jnp.concatenate([tar_values, zf])
    sv = jnp.concatenate([src_values, zf])

    y0 = _tc_normalize(x0)
    q = _sc_pass(y0, x0, t0, t1, tv, zeros)
    mt = _tc_sum2(q[0], q[1])
    p = _sc_pass(y0, mt, s0, s1, sv, zeros)
    x1, y1 = _tc_combine_norm(p[0], p[1], x0)

    q = _sc_pass(y1, x1, t0, t1, tv, zeros)
    mt = _tc_sum2(q[0], q[1])
    p = _sc_pass(y1, mt, s0, s1, sv, zeros)
    return _tc_final(p[0], p[1], x0, x1)
